# baseline (speedup 1.0000x reference)
import jax
import jax.numpy as jnp
from jax import lax
from jax.experimental import pallas as pl
from jax.experimental.pallas import tpu as pltpu
from jax.experimental.pallas import tpu_sc as plsc

_N = 10000
_E = 320000
_DIN = 128
_DH = 128
_DOUT = 64

_NC = 2
_NS = 16
_NW = _NC * _NS
_CHUNK = 128
_EPW = _E // _NW
_G = 8
_NGRP = 10
_NCHUNK = _G * _NGRP
_EPW_PAD = _NCHUNK * _CHUNK
_NP = 10240
_RPT = _NP // _NS
_BLK = 10000


def _dotT(a, w):
    return lax.dot_general(a, w, (((1,), (1,)), ((), ())),
                           preferred_element_type=jnp.float32)


def _proj2(x, Wl, Wr, b, dout):
    n, din = x.shape

    def body(x_ref, wl_ref, wr_ref, b_ref, yl_ref, yr_ref):
        xv = x_ref[...]
        yl_ref[...] = _dotT(xv, wl_ref[...])
        yr_ref[...] = _dotT(xv, wr_ref[...]) + b_ref[...]

    return pl.pallas_call(
        body,
        grid=(n // _BLK,),
        in_specs=[
            pl.BlockSpec((_BLK, din), lambda i: (i, 0)),
            pl.BlockSpec(Wl.shape, lambda i: (0, 0)),
            pl.BlockSpec(Wr.shape, lambda i: (0, 0)),
            pl.BlockSpec((1, dout), lambda i: (0, 0)),
        ],
        out_specs=[
            pl.BlockSpec((_BLK, dout), lambda i: (i, 0)),
            pl.BlockSpec((_BLK, dout), lambda i: (i, 0)),
        ],
        out_shape=[
            jax.ShapeDtypeStruct((n, dout), jnp.float32),
            jax.ShapeDtypeStruct((n, dout), jnp.float32),
        ],
    )(x, Wl, Wr, b.reshape(1, -1))


def _combine_proj(parts, degp, yr, Wl, Wr, b, dout):
    n, d = yr.shape

    def body(p_ref, g_ref, yr_ref, wl_ref, wr_ref, b_ref, yl_ref, yr2_ref):
        p = p_ref[0] + p_ref[1]
        deg = g_ref[0, :, 0:1] + g_ref[1, :, 0:1]
        inv = 1.0 / jnp.maximum(deg, 1.0)
        h = jnp.maximum(p * inv + yr_ref[...], 0.0)
        yl_ref[...] = _dotT(h, wl_ref[...])
        yr2_ref[...] = _dotT(h, wr_ref[...]) + b_ref[...]

    return pl.pallas_call(
        body,
        grid=(n // _BLK,),
        in_specs=[
            pl.BlockSpec((2, _BLK, d), lambda i: (0, i, 0)),
            pl.BlockSpec((2, _BLK, degp.shape[2]), lambda i: (0, i, 0)),
            pl.BlockSpec((_BLK, d), lambda i: (i, 0)),
            pl.BlockSpec(Wl.shape, lambda i: (0, 0)),
            pl.BlockSpec(Wr.shape, lambda i: (0, 0)),
            pl.BlockSpec((1, dout), lambda i: (0, 0)),
        ],
        out_specs=[
            pl.BlockSpec((_BLK, dout), lambda i: (i, 0)),
            pl.BlockSpec((_BLK, dout), lambda i: (i, 0)),
        ],
        out_shape=[
            jax.ShapeDtypeStruct((n, dout), jnp.float32),
            jax.ShapeDtypeStruct((n, dout), jnp.float32),
        ],
    )(parts, degp, yr, Wl, Wr, b.reshape(1, -1))


def _finalize(parts, degp, yr):
    n, d = yr.shape

    def body(p_ref, g_ref, yr_ref, o_ref):
        p = p_ref[0, :, 0:d] + p_ref[1, :, 0:d]
        deg = g_ref[0, :, 0:1] + g_ref[1, :, 0:1]
        inv = 1.0 / jnp.maximum(deg, 1.0)
        z = p * inv + yr_ref[...]
        m = jnp.max(z, axis=1, keepdims=True)
        e = jnp.exp(z - m)
        o_ref[...] = z - m - jnp.log(jnp.sum(e, axis=1, keepdims=True))

    return pl.pallas_call(
        body,
        grid=(n // _BLK,),
        in_specs=[
            pl.BlockSpec((2, _BLK, parts.shape[2]), lambda i: (0, i, 0)),
            pl.BlockSpec((2, _BLK, degp.shape[2]), lambda i: (0, i, 0)),
            pl.BlockSpec((_BLK, d), lambda i: (i, 0)),
        ],
        out_specs=pl.BlockSpec((_BLK, d), lambda i: (i, 0)),
        out_shape=jax.ShapeDtypeStruct((n, d), jnp.float32),
    )(parts, degp, yr)


_MESH = plsc.VectorSubcoreMesh(core_axis_name="c", subcore_axis_name="s")


def _sc_segsum(d):
    out_type = jax.ShapeDtypeStruct((_NC, _NP, d), jnp.float32)
    scratch = [
        pltpu.VMEM((2, _G, _CHUNK), jnp.int32),
        pltpu.VMEM((2, _G, _CHUNK), jnp.int32),
        pltpu.VMEM((_CHUNK, d), jnp.float32),
        pltpu.VMEM((_CHUNK, d), jnp.float32),
        pltpu.VMEM_SHARED((_NP, d), jnp.float32),
        pltpu.SemaphoreType.DMA,
        pltpu.SemaphoreType.DMA,
        pltpu.SemaphoreType.DMA,
        pltpu.SemaphoreType.DMA,
        pltpu.SemaphoreType.DMA,
    ]

    def body(y_hbm, srcp, dstp, zrow, out_hbm,
             srcg, dstg, r0, r1, acc, g0, g1, s0, s1, isem):
        c = lax.axis_index("c")
        s = lax.axis_index("s")
        w = s * _NC + c
        pltpu.sync_copy(zrow, acc.at[pl.ds(s * _RPT, _RPT)])
        pltpu.sync_copy(srcp.at[w, pl.ds(0, _G)], srcg.at[0])
        pltpu.sync_copy(dstp.at[w, pl.ds(0, _G)], dstg.at[0])
        plsc.subcore_barrier()

        def group(g, carry):
            gb = g % 2
            nb = (g + 1) % 2

            @pl.when(g > 0)
            def _drain_idx():
                pltpu.make_async_copy(
                    srcp.at[w, pl.ds(g * _G, _G)], srcg.at[gb], isem).wait()
                pltpu.make_async_copy(
                    dstp.at[w, pl.ds(g * _G, _G)], dstg.at[gb], isem).wait()

            gd = [pltpu.async_copy(y_hbm.at[srcg.at[gb, 0]], r0, g0),
                  pltpu.async_copy(y_hbm.at[srcg.at[gb, 1]], r1, g1)]

            @pl.when(g + 1 < _NGRP)
            def _stage_next():
                pltpu.async_copy(srcp.at[w, pl.ds((g + 1) * _G, _G)],
                                 srcg.at[nb], isem)
                pltpu.async_copy(dstp.at[w, pl.ds((g + 1) * _G, _G)],
                                 dstg.at[nb], isem)

            sd = [None, None]
            for k in range(_G):
                r, gs, ss = (r0, g0, s0) if k % 2 == 0 else (r1, g1, s1)
                b = k % 2
                gd[b].wait()
                sd[b] = pltpu.async_copy(r, acc.at[dstg.at[gb, k]], ss,
                                         add=True)
                if k + 2 < _G:
                    sd[b].wait()
                    gd[b] = pltpu.async_copy(y_hbm.at[srcg.at[gb, k + 2]],
                                             r, gs)
            sd[0].wait()
            sd[1].wait()
            return carry

        lax.fori_loop(0, _NGRP, group, 0)
        plsc.subcore_barrier()
        pltpu.sync_copy(acc.at[pl.ds(s * _RPT, _RPT)],
                        out_hbm.at[c, pl.ds(s * _RPT, _RPT)])

    return pl.kernel(body, out_type=out_type, mesh=_MESH, scratch_types=scratch)


def _sc_deg(dstp, z128, ones):
    scratch = [
        pltpu.VMEM((_NCHUNK, _CHUNK), jnp.int32),
        pltpu.VMEM((_CHUNK, _DH), jnp.float32),
        pltpu.VMEM_SHARED((_NP, _DH), jnp.float32),
        pltpu.SemaphoreType.DMA,
    ]

    def body(dstp_hbm, z16_hbm, ones_hbm, deg_hbm, dst_v, ones_v, dacc, ssem):
        c = lax.axis_index("c")
        s = lax.axis_index("s")
        w = s * _NC + c
        pltpu.sync_copy(z16_hbm, dacc.at[pl.ds(s * _RPT, _RPT)])
        pltpu.sync_copy(dstp_hbm.at[w], dst_v)
        pltpu.sync_copy(ones_hbm, ones_v)
        plsc.subcore_barrier()

        def group(g, carry):
            sd = [pltpu.async_copy(ones_v, dacc.at[dst_v.at[g * _G + k]],
                                   ssem, add=True)
                  for k in range(_G)]
            for d_ in sd:
                d_.wait()
            return carry

        lax.fori_loop(0, _NGRP, group, 0)
        plsc.subcore_barrier()
        pltpu.sync_copy(dacc.at[pl.ds(s * _RPT, _RPT)],
                        deg_hbm.at[c, pl.ds(s * _RPT, _RPT)])

    return pl.kernel(
        body,
        out_type=jax.ShapeDtypeStruct((_NC, _NP, _DH), jnp.float32),
        mesh=_MESH,
        scratch_types=scratch,
    )(dstp, z128, ones)


def _sc_probe(d, mode):
    out_type = jax.ShapeDtypeStruct((_NC, _NP, d), jnp.float32)
    scratch = [
        pltpu.VMEM((_NCHUNK, _CHUNK), jnp.int32),
        pltpu.VMEM((_CHUNK, d), jnp.float32),
        pltpu.VMEM((_CHUNK, d), jnp.float32),
        pltpu.VMEM_SHARED((_NP, d), jnp.float32),
        pltpu.SemaphoreType.DMA,
        pltpu.SemaphoreType.DMA,
    ]

    def body(y_hbm, srcp, dstp, zrow, out_hbm, src_v, r0, r1, acc, g0, g1):
        c = lax.axis_index("c")
        s = lax.axis_index("s")
        w = s * _NC + c
        pltpu.sync_copy(zrow, acc.at[pl.ds(s * _RPT, _RPT)])
        pltpu.sync_copy(srcp.at[w], src_v)
        plsc.subcore_barrier()

        if mode == "g1":
            def step(j, carry):
                pltpu.async_copy(y_hbm.at[src_v.at[j]], r0, g0).wait()
                return carry
            lax.fori_loop(0, _NCHUNK, step, 0)
        elif mode == "g2":
            d0 = pltpu.async_copy(y_hbm.at[src_v.at[0]], r0, g0)
            d1 = pltpu.async_copy(y_hbm.at[src_v.at[1]], r1, g1)

            def pair(i, carry):
                d0 = pltpu.make_async_copy(y_hbm.at[src_v.at[2 * i]], r0, g0)
                d1 = pltpu.make_async_copy(y_hbm.at[src_v.at[2 * i + 1]],
                                           r1, g1)
                d0.wait()
                @pl.when(i + 1 < _NCHUNK // 2)
                def _():
                    pltpu.async_copy(y_hbm.at[src_v.at[2 * i + 2]], r0, g0)
                d1.wait()
                @pl.when(i + 1 < _NCHUNK // 2)
                def _():
                    pltpu.async_copy(y_hbm.at[src_v.at[2 * i + 3]], r1, g1)
                return carry
            lax.fori_loop(0, _NCHUNK // 2, pair, 0)
        elif mode == "lin":
            def step(j, carry):
                base = ((w * _NCHUNK + j) % 77) * _CHUNK
                pltpu.sync_copy(y_hbm.at[pl.ds(base, _CHUNK)], r0)
                return carry
            lax.fori_loop(0, _NCHUNK, step, 0)

        plsc.subcore_barrier()
        pltpu.sync_copy(acc.at[pl.ds(s * _RPT, _RPT)],
                        out_hbm.at[c, pl.ds(s * _RPT, _RPT)])

    return pl.kernel(body, out_type=out_type, mesh=_MESH, scratch_types=scratch)


def kernel(x, edge_index, Wl1, Wr1, b1, Wl2, Wr2, b2, Wl3, Wr3, b3):
    src = edge_index[0].reshape(_NW, _EPW)
    dst = edge_index[1].reshape(_NW, _EPW)
    npad = _EPW_PAD - _EPW
    pad_i = jnp.arange(npad, dtype=jnp.int32)
    pad_s = jnp.broadcast_to((pad_i * 41) % _N, (_NW, npad))
    pad_d = jnp.broadcast_to(_N + (pad_i % (_NP - _N)), (_NW, npad))
    srcp = jnp.concatenate([src, pad_s], axis=1).reshape(_NW, _NCHUNK, _CHUNK)
    dstp = jnp.concatenate([dst, pad_d], axis=1).reshape(_NW, _NCHUNK, _CHUNK)
    z128 = jnp.zeros((_RPT, _DH), jnp.float32)
    ones = jnp.ones((_CHUNK, _DH), jnp.float32)

    degp = _sc_deg(dstp, z128, ones)
    yl1, yr1 = _proj2(x, Wl1, Wr1, b1, _DH)
    parts1 = _sc_segsum(_DH)(yl1, srcp, dstp, z128)
    yl2, yr2 = _combine_proj(parts1, degp, yr1, Wl2, Wr2, b2, _DH)
    parts2 = _sc_segsum(_DH)(yl2, srcp, dstp, z128)
    wpad = jnp.zeros((_DH - _DOUT, _DH), jnp.float32)
    Wl3p = jnp.concatenate([Wl3, wpad], axis=0)
    Wr3p = jnp.concatenate([Wr3, wpad], axis=0)
    b3p = jnp.concatenate([b3, jnp.zeros((_DH - _DOUT,), jnp.float32)])
    yl3, yr3p = _combine_proj(parts2, degp, yr2, Wl3p, Wr3p, b3p, _DH)
    yr3 = yr3p[:, :_DOUT]
    parts3 = _sc_segsum(_DH)(yl3, srcp, dstp, z128)
    out = _finalize(parts3, degp, yr3)
    return out


# speedup vs baseline: 1.0017x; 1.0017x over previous
import jax
import jax.numpy as jnp
from jax import lax
from jax.experimental import pallas as pl
from jax.experimental.pallas import tpu as pltpu
from jax.experimental.pallas import tpu_sc as plsc

_N = 10000
_E = 320000
_DIN = 128
_DH = 128
_DOUT = 64

_NC = 2
_NS = 16
_NW = _NC * _NS
_CHUNK = 128
_EPW = _E // _NW
_G = 8
_NGRP = 10
_NCHUNK = _G * _NGRP
_EPW_PAD = _NCHUNK * _CHUNK
_NP = 10240
_RPT = _NP // _NS
_BLK = 2000


def _dotT(a, w):
    return lax.dot_general(a, w, (((1,), (1,)), ((), ())),
                           preferred_element_type=jnp.float32)


def _proj2(x, Wl, Wr, b, dout):
    n, din = x.shape

    def body(x_ref, wl_ref, wr_ref, b_ref, yl_ref, yr_ref):
        xv = x_ref[...]
        yl_ref[...] = _dotT(xv, wl_ref[...])
        yr_ref[...] = _dotT(xv, wr_ref[...]) + b_ref[...]

    return pl.pallas_call(
        body,
        grid=(n // _BLK,),
        in_specs=[
            pl.BlockSpec((_BLK, din), lambda i: (i, 0)),
            pl.BlockSpec(Wl.shape, lambda i: (0, 0)),
            pl.BlockSpec(Wr.shape, lambda i: (0, 0)),
            pl.BlockSpec((1, dout), lambda i: (0, 0)),
        ],
        out_specs=[
            pl.BlockSpec((_BLK, dout), lambda i: (i, 0)),
            pl.BlockSpec((_BLK, dout), lambda i: (i, 0)),
        ],
        out_shape=[
            jax.ShapeDtypeStruct((n, dout), jnp.float32),
            jax.ShapeDtypeStruct((n, dout), jnp.float32),
        ],
    )(x, Wl, Wr, b.reshape(1, -1))


def _combine_proj(parts, degp, yr, Wl, Wr, b, dout):
    n, d = yr.shape

    def body(p_ref, g_ref, yr_ref, wl_ref, wr_ref, b_ref, yl_ref, yr2_ref):
        p = p_ref[0] + p_ref[1]
        deg = g_ref[0, :, 0:1] + g_ref[1, :, 0:1]
        inv = 1.0 / jnp.maximum(deg, 1.0)
        h = jnp.maximum(p * inv + yr_ref[...], 0.0)
        yl_ref[...] = _dotT(h, wl_ref[...])
        yr2_ref[...] = _dotT(h, wr_ref[...]) + b_ref[...]

    return pl.pallas_call(
        body,
        grid=(n // _BLK,),
        in_specs=[
            pl.BlockSpec((2, _BLK, d), lambda i: (0, i, 0)),
            pl.BlockSpec((2, _BLK, degp.shape[2]), lambda i: (0, i, 0)),
            pl.BlockSpec((_BLK, d), lambda i: (i, 0)),
            pl.BlockSpec(Wl.shape, lambda i: (0, 0)),
            pl.BlockSpec(Wr.shape, lambda i: (0, 0)),
            pl.BlockSpec((1, dout), lambda i: (0, 0)),
        ],
        out_specs=[
            pl.BlockSpec((_BLK, dout), lambda i: (i, 0)),
            pl.BlockSpec((_BLK, dout), lambda i: (i, 0)),
        ],
        out_shape=[
            jax.ShapeDtypeStruct((n, dout), jnp.float32),
            jax.ShapeDtypeStruct((n, dout), jnp.float32),
        ],
    )(parts, degp, yr, Wl, Wr, b.reshape(1, -1))


def _finalize(parts, degp, yr):
    n, d = yr.shape

    def body(p_ref, g_ref, yr_ref, o_ref):
        p = p_ref[0, :, 0:d] + p_ref[1, :, 0:d]
        deg = g_ref[0, :, 0:1] + g_ref[1, :, 0:1]
        inv = 1.0 / jnp.maximum(deg, 1.0)
        z = p * inv + yr_ref[...]
        m = jnp.max(z, axis=1, keepdims=True)
        e = jnp.exp(z - m)
        o_ref[...] = z - m - jnp.log(jnp.sum(e, axis=1, keepdims=True))

    return pl.pallas_call(
        body,
        grid=(n // _BLK,),
        in_specs=[
            pl.BlockSpec((2, _BLK, parts.shape[2]), lambda i: (0, i, 0)),
            pl.BlockSpec((2, _BLK, degp.shape[2]), lambda i: (0, i, 0)),
            pl.BlockSpec((_BLK, d), lambda i: (i, 0)),
        ],
        out_specs=pl.BlockSpec((_BLK, d), lambda i: (i, 0)),
        out_shape=jax.ShapeDtypeStruct((n, d), jnp.float32),
    )(parts, degp, yr)


_MESH = plsc.VectorSubcoreMesh(core_axis_name="c", subcore_axis_name="s")


def _sc_segsum(d):
    out_type = jax.ShapeDtypeStruct((_NC, _NP, d), jnp.float32)
    scratch = [
        pltpu.VMEM((2, _G, _CHUNK), jnp.int32),
        pltpu.VMEM((2, _G, _CHUNK), jnp.int32),
        pltpu.VMEM((_CHUNK, d), jnp.float32),
        pltpu.VMEM((_CHUNK, d), jnp.float32),
        pltpu.VMEM_SHARED((_NP, d), jnp.float32),
        pltpu.SemaphoreType.DMA,
        pltpu.SemaphoreType.DMA,
        pltpu.SemaphoreType.DMA,
        pltpu.SemaphoreType.DMA,
        pltpu.SemaphoreType.DMA,
    ]

    def body(y_hbm, srcp, dstp, zrow, out_hbm,
             srcg, dstg, r0, r1, acc, g0, g1, s0, s1, isem):
        c = lax.axis_index("c")
        s = lax.axis_index("s")
        w = s * _NC + c
        pltpu.sync_copy(zrow, acc.at[pl.ds(s * _RPT, _RPT)])
        pltpu.sync_copy(srcp.at[w, pl.ds(0, _G)], srcg.at[0])
        pltpu.sync_copy(dstp.at[w, pl.ds(0, _G)], dstg.at[0])
        plsc.subcore_barrier()

        def group(g, carry):
            gb = g % 2
            nb = (g + 1) % 2

            @pl.when(g > 0)
            def _drain_idx():
                pltpu.make_async_copy(
                    srcp.at[w, pl.ds(g * _G, _G)], srcg.at[gb], isem).wait()
                pltpu.make_async_copy(
                    dstp.at[w, pl.ds(g * _G, _G)], dstg.at[gb], isem).wait()

            gd = [pltpu.async_copy(y_hbm.at[srcg.at[gb, 0]], r0, g0),
                  pltpu.async_copy(y_hbm.at[srcg.at[gb, 1]], r1, g1)]

            @pl.when(g + 1 < _NGRP)
            def _stage_next():
                pltpu.async_copy(srcp.at[w, pl.ds((g + 1) * _G, _G)],
                                 srcg.at[nb], isem)
                pltpu.async_copy(dstp.at[w, pl.ds((g + 1) * _G, _G)],
                                 dstg.at[nb], isem)

            sd = [None, None]
            for k in range(_G):
                r, gs, ss = (r0, g0, s0) if k % 2 == 0 else (r1, g1, s1)
                b = k % 2
                gd[b].wait()
                sd[b] = pltpu.async_copy(r, acc.at[dstg.at[gb, k]], ss,
                                         add=True)
                if k + 2 < _G:
                    sd[b].wait()
                    gd[b] = pltpu.async_copy(y_hbm.at[srcg.at[gb, k + 2]],
                                             r, gs)
            sd[0].wait()
            sd[1].wait()
            return carry

        lax.fori_loop(0, _NGRP, group, 0)
        plsc.subcore_barrier()
        pltpu.sync_copy(acc.at[pl.ds(s * _RPT, _RPT)],
                        out_hbm.at[c, pl.ds(s * _RPT, _RPT)])

    return pl.kernel(body, out_type=out_type, mesh=_MESH, scratch_types=scratch)


def _sc_deg(dstp, z128, ones):
    scratch = [
        pltpu.VMEM((_NCHUNK, _CHUNK), jnp.int32),
        pltpu.VMEM((_CHUNK, _DH), jnp.float32),
        pltpu.VMEM_SHARED((_NP, _DH), jnp.float32),
        pltpu.SemaphoreType.DMA,
    ]

    def body(dstp_hbm, z16_hbm, ones_hbm, deg_hbm, dst_v, ones_v, dacc, ssem):
        c = lax.axis_index("c")
        s = lax.axis_index("s")
        w = s * _NC + c
        pltpu.sync_copy(z16_hbm, dacc.at[pl.ds(s * _RPT, _RPT)])
        pltpu.sync_copy(dstp_hbm.at[w], dst_v)
        pltpu.sync_copy(ones_hbm, ones_v)
        plsc.subcore_barrier()

        def group(g, carry):
            sd = [pltpu.async_copy(ones_v, dacc.at[dst_v.at[g * _G + k]],
                                   ssem, add=True)
                  for k in range(_G)]
            for d_ in sd:
                d_.wait()
            return carry

        lax.fori_loop(0, _NGRP, group, 0)
        plsc.subcore_barrier()
        pltpu.sync_copy(dacc.at[pl.ds(s * _RPT, _RPT)],
                        deg_hbm.at[c, pl.ds(s * _RPT, _RPT)])

    return pl.kernel(
        body,
        out_type=jax.ShapeDtypeStruct((_NC, _NP, _DH), jnp.float32),
        mesh=_MESH,
        scratch_types=scratch,
    )(dstp, z128, ones)


def _sc_probe(d, mode):
    out_type = jax.ShapeDtypeStruct((_NC, _NP, d), jnp.float32)
    scratch = [
        pltpu.VMEM((_NCHUNK, _CHUNK), jnp.int32),
        pltpu.VMEM((_CHUNK, d), jnp.float32),
        pltpu.VMEM((_CHUNK, d), jnp.float32),
        pltpu.VMEM_SHARED((_NP, d), jnp.float32),
        pltpu.SemaphoreType.DMA,
        pltpu.SemaphoreType.DMA,
    ]

    def body(y_hbm, srcp, dstp, zrow, out_hbm, src_v, r0, r1, acc, g0, g1):
        c = lax.axis_index("c")
        s = lax.axis_index("s")
        w = s * _NC + c
        pltpu.sync_copy(zrow, acc.at[pl.ds(s * _RPT, _RPT)])
        pltpu.sync_copy(srcp.at[w], src_v)
        plsc.subcore_barrier()

        if mode == "g1":
            def step(j, carry):
                pltpu.async_copy(y_hbm.at[src_v.at[j]], r0, g0).wait()
                return carry
            lax.fori_loop(0, _NCHUNK, step, 0)
        elif mode == "g2":
            d0 = pltpu.async_copy(y_hbm.at[src_v.at[0]], r0, g0)
            d1 = pltpu.async_copy(y_hbm.at[src_v.at[1]], r1, g1)

            def pair(i, carry):
                d0 = pltpu.make_async_copy(y_hbm.at[src_v.at[2 * i]], r0, g0)
                d1 = pltpu.make_async_copy(y_hbm.at[src_v.at[2 * i + 1]],
                                           r1, g1)
                d0.wait()
                @pl.when(i + 1 < _NCHUNK // 2)
                def _():
                    pltpu.async_copy(y_hbm.at[src_v.at[2 * i + 2]], r0, g0)
                d1.wait()
                @pl.when(i + 1 < _NCHUNK // 2)
                def _():
                    pltpu.async_copy(y_hbm.at[src_v.at[2 * i + 3]], r1, g1)
                return carry
            lax.fori_loop(0, _NCHUNK // 2, pair, 0)
        elif mode == "lin":
            def step(j, carry):
                base = ((w * _NCHUNK + j) % 77) * _CHUNK
                pltpu.sync_copy(y_hbm.at[pl.ds(base, _CHUNK)], r0)
                return carry
            lax.fori_loop(0, _NCHUNK, step, 0)

        plsc.subcore_barrier()
        pltpu.sync_copy(acc.at[pl.ds(s * _RPT, _RPT)],
                        out_hbm.at[c, pl.ds(s * _RPT, _RPT)])

    return pl.kernel(body, out_type=out_type, mesh=_MESH, scratch_types=scratch)


def kernel(x, edge_index, Wl1, Wr1, b1, Wl2, Wr2, b2, Wl3, Wr3, b3):
    src = edge_index[0].reshape(_NW, _EPW)
    dst = edge_index[1].reshape(_NW, _EPW)
    npad = _EPW_PAD - _EPW
    pad_i = jnp.arange(npad, dtype=jnp.int32)
    pad_s = jnp.broadcast_to((pad_i * 41) % _N, (_NW, npad))
    pad_d = jnp.broadcast_to(_N + (pad_i % (_NP - _N)), (_NW, npad))
    srcp = jnp.concatenate([src, pad_s], axis=1).reshape(_NW, _NCHUNK, _CHUNK)
    dstp = jnp.concatenate([dst, pad_d], axis=1).reshape(_NW, _NCHUNK, _CHUNK)
    z128 = jnp.zeros((_RPT, _DH), jnp.float32)
    ones = jnp.ones((_CHUNK, _DH), jnp.float32)

    degp = _sc_deg(dstp, z128, ones)
    yl1, yr1 = _proj2(x, Wl1, Wr1, b1, _DH)
    parts1 = _sc_segsum(_DH)(yl1, srcp, dstp, z128)
    yl2, yr2 = _combine_proj(parts1, degp, yr1, Wl2, Wr2, b2, _DH)
    parts2 = _sc_segsum(_DH)(yl2, srcp, dstp, z128)
    wpad = jnp.zeros((_DH - _DOUT, _DH), jnp.float32)
    Wl3p = jnp.concatenate([Wl3, wpad], axis=0)
    Wr3p = jnp.concatenate([Wr3, wpad], axis=0)
    b3p = jnp.concatenate([b3, jnp.zeros((_DH - _DOUT,), jnp.float32)])
    yl3, yr3p = _combine_proj(parts2, degp, yr2, Wl3p, Wr3p, b3p, _DH)
    yr3 = yr3p[:, :_DOUT]
    parts3 = _sc_segsum(_DH)(yl3, srcp, dstp, z128)
    out = _finalize(parts3, degp, yr3)
    return out


# speedup vs baseline: 1.0043x; 1.0026x over previous
import jax
import jax.numpy as jnp
from jax import lax
from jax.experimental import pallas as pl
from jax.experimental.pallas import tpu as pltpu
from jax.experimental.pallas import tpu_sc as plsc

_N = 10000
_E = 320000
_DIN = 128
_DH = 128
_DOUT = 64

_NC = 2
_NS = 16
_NW = _NC * _NS
_CHUNK = 128
_EPW = _E // _NW
_G = 8
_NGRP = 10
_NCHUNK = _G * _NGRP
_EPW_PAD = _NCHUNK * _CHUNK
_NP = 10240
_RPT = _NP // _NS
_BLK = 2000


def _dotT(a, w):
    return lax.dot_general(a, w, (((1,), (1,)), ((), ())),
                           preferred_element_type=jnp.float32)


def _proj2(x, Wl, Wr, b, dout):
    n, din = x.shape

    def body(x_ref, wl_ref, wr_ref, b_ref, yl_ref, yr_ref):
        xv = x_ref[...]
        yl_ref[...] = _dotT(xv, wl_ref[...])
        yr_ref[...] = _dotT(xv, wr_ref[...]) + b_ref[...]

    return pl.pallas_call(
        body,
        grid=(n // _BLK,),
        in_specs=[
            pl.BlockSpec((_BLK, din), lambda i: (i, 0)),
            pl.BlockSpec(Wl.shape, lambda i: (0, 0)),
            pl.BlockSpec(Wr.shape, lambda i: (0, 0)),
            pl.BlockSpec((1, dout), lambda i: (0, 0)),
        ],
        out_specs=[
            pl.BlockSpec((_BLK, dout), lambda i: (i, 0)),
            pl.BlockSpec((_BLK, dout), lambda i: (i, 0)),
        ],
        out_shape=[
            jax.ShapeDtypeStruct((n, dout), jnp.float32),
            jax.ShapeDtypeStruct((n, dout), jnp.float32),
        ],
    )(x, Wl, Wr, b.reshape(1, -1))


def _combine_proj(parts, degp, yr, Wl, Wr, b, dout):
    n, d = yr.shape

    def body(p_ref, g_ref, yr_ref, wl_ref, wr_ref, b_ref, yl_ref, yr2_ref):
        p = p_ref[0] + p_ref[1]
        deg = g_ref[0, :, 0:1] + g_ref[1, :, 0:1]
        inv = 1.0 / jnp.maximum(deg, 1.0)
        h = jnp.maximum(p * inv + yr_ref[...], 0.0)
        yl_ref[...] = _dotT(h, wl_ref[...])
        yr2_ref[...] = _dotT(h, wr_ref[...]) + b_ref[...]

    return pl.pallas_call(
        body,
        grid=(n // _BLK,),
        in_specs=[
            pl.BlockSpec((2, _BLK, d), lambda i: (0, i, 0)),
            pl.BlockSpec((2, _BLK, degp.shape[2]), lambda i: (0, i, 0)),
            pl.BlockSpec((_BLK, d), lambda i: (i, 0)),
            pl.BlockSpec(Wl.shape, lambda i: (0, 0)),
            pl.BlockSpec(Wr.shape, lambda i: (0, 0)),
            pl.BlockSpec((1, dout), lambda i: (0, 0)),
        ],
        out_specs=[
            pl.BlockSpec((_BLK, dout), lambda i: (i, 0)),
            pl.BlockSpec((_BLK, dout), lambda i: (i, 0)),
        ],
        out_shape=[
            jax.ShapeDtypeStruct((n, dout), jnp.float32),
            jax.ShapeDtypeStruct((n, dout), jnp.float32),
        ],
    )(parts, degp, yr, Wl, Wr, b.reshape(1, -1))


def _finalize(parts, degp, yr):
    n, d = yr.shape

    def body(p_ref, g_ref, yr_ref, o_ref):
        p = p_ref[0, :, 0:d] + p_ref[1, :, 0:d]
        deg = g_ref[0, :, 0:1] + g_ref[1, :, 0:1]
        inv = 1.0 / jnp.maximum(deg, 1.0)
        z = p * inv + yr_ref[...]
        m = jnp.max(z, axis=1, keepdims=True)
        e = jnp.exp(z - m)
        o_ref[...] = z - m - jnp.log(jnp.sum(e, axis=1, keepdims=True))

    return pl.pallas_call(
        body,
        grid=(n // _BLK,),
        in_specs=[
            pl.BlockSpec((2, _BLK, parts.shape[2]), lambda i: (0, i, 0)),
            pl.BlockSpec((2, _BLK, degp.shape[2]), lambda i: (0, i, 0)),
            pl.BlockSpec((_BLK, d), lambda i: (i, 0)),
        ],
        out_specs=pl.BlockSpec((_BLK, d), lambda i: (i, 0)),
        out_shape=jax.ShapeDtypeStruct((n, d), jnp.float32),
    )(parts, degp, yr)


_MESH = plsc.VectorSubcoreMesh(core_axis_name="c", subcore_axis_name="s")


def _sc_segsum(d):
    out_type = jax.ShapeDtypeStruct((_NC, _NP, d), jnp.float32)
    scratch = [
        pltpu.VMEM((2, _G, _CHUNK), jnp.int32),
        pltpu.VMEM((2, _G, _CHUNK), jnp.int32),
        pltpu.VMEM((_CHUNK, d), jnp.float32),
        pltpu.VMEM((_CHUNK, d), jnp.float32),
        pltpu.VMEM_SHARED((_NP, d), jnp.float32),
        pltpu.SemaphoreType.DMA,
        pltpu.SemaphoreType.DMA,
        pltpu.SemaphoreType.DMA,
        pltpu.SemaphoreType.DMA,
        pltpu.SemaphoreType.DMA,
    ]

    def body(y_hbm, srcp, dstp, zrow, out_hbm,
             srcg, dstg, r0, r1, acc, g0, g1, s0, s1, isem):
        c = lax.axis_index("c")
        s = lax.axis_index("s")
        w = s * _NC + c
        pltpu.sync_copy(zrow, acc.at[pl.ds(s * _RPT, _RPT)])
        pltpu.sync_copy(srcp.at[w, pl.ds(0, _G)], srcg.at[0])
        pltpu.sync_copy(dstp.at[w, pl.ds(0, _G)], dstg.at[0])
        plsc.subcore_barrier()

        def group(g, carry):
            gb = g % 2
            nb = (g + 1) % 2

            @pl.when(g > 0)
            def _drain_idx():
                pltpu.make_async_copy(
                    srcp.at[w, pl.ds(g * _G, _G)], srcg.at[gb], isem).wait()
                pltpu.make_async_copy(
                    dstp.at[w, pl.ds(g * _G, _G)], dstg.at[gb], isem).wait()

            gd = [pltpu.async_copy(y_hbm.at[srcg.at[gb, 0]], r0, g0),
                  pltpu.async_copy(y_hbm.at[srcg.at[gb, 1]], r1, g1)]

            @pl.when(g + 1 < _NGRP)
            def _stage_next():
                pltpu.async_copy(srcp.at[w, pl.ds((g + 1) * _G, _G)],
                                 srcg.at[nb], isem)
                pltpu.async_copy(dstp.at[w, pl.ds((g + 1) * _G, _G)],
                                 dstg.at[nb], isem)

            sd = [None, None]
            for k in range(_G):
                r, gs, ss = (r0, g0, s0) if k % 2 == 0 else (r1, g1, s1)
                b = k % 2
                gd[b].wait()
                sd[b] = pltpu.async_copy(r, acc.at[dstg.at[gb, k]], ss,
                                         add=True)
                if k + 2 < _G:
                    sd[b].wait()
                    gd[b] = pltpu.async_copy(y_hbm.at[srcg.at[gb, k + 2]],
                                             r, gs)
            sd[0].wait()
            sd[1].wait()
            return carry

        lax.fori_loop(0, _NGRP, group, 0)
        plsc.subcore_barrier()
        pltpu.sync_copy(acc.at[pl.ds(s * _RPT, _RPT)],
                        out_hbm.at[c, pl.ds(s * _RPT, _RPT)])

    return pl.kernel(body, out_type=out_type, mesh=_MESH, scratch_types=scratch)


def _sc_deg(dstp, z128, ones):
    scratch = [
        pltpu.VMEM((_NCHUNK, _CHUNK), jnp.int32),
        pltpu.VMEM((_CHUNK, _DH), jnp.float32),
        pltpu.VMEM_SHARED((_NP, _DH), jnp.float32),
        pltpu.SemaphoreType.DMA,
    ]

    def body(dstp_hbm, z16_hbm, ones_hbm, deg_hbm, dst_v, ones_v, dacc, ssem):
        c = lax.axis_index("c")
        s = lax.axis_index("s")
        w = s * _NC + c
        pltpu.sync_copy(z16_hbm, dacc.at[pl.ds(s * _RPT, _RPT)])
        pltpu.sync_copy(dstp_hbm.at[w], dst_v)
        pltpu.sync_copy(ones_hbm, ones_v)
        plsc.subcore_barrier()

        def group(g, carry):
            sd = [pltpu.async_copy(ones_v, dacc.at[dst_v.at[g * _G + k]],
                                   ssem, add=True)
                  for k in range(_G)]
            for d_ in sd:
                d_.wait()
            return carry

        lax.fori_loop(0, _NGRP, group, 0)
        plsc.subcore_barrier()
        pltpu.sync_copy(dacc.at[pl.ds(s * _RPT, _RPT)],
                        deg_hbm.at[c, pl.ds(s * _RPT, _RPT)])

    return pl.kernel(
        body,
        out_type=jax.ShapeDtypeStruct((_NC, _NP, _DH), jnp.float32),
        mesh=_MESH,
        scratch_types=scratch,
    )(dstp, z128, ones)


def kernel(x, edge_index, Wl1, Wr1, b1, Wl2, Wr2, b2, Wl3, Wr3, b3):
    src = edge_index[0].reshape(_NW, _EPW)
    dst = edge_index[1].reshape(_NW, _EPW)
    npad = _EPW_PAD - _EPW
    pad_i = jnp.arange(npad, dtype=jnp.int32)
    pad_s = jnp.broadcast_to((pad_i * 41) % _N, (_NW, npad))
    pad_d = jnp.broadcast_to(_N + (pad_i % (_NP - _N)), (_NW, npad))
    srcp = jnp.concatenate([src, pad_s], axis=1).reshape(_NW, _NCHUNK, _CHUNK)
    dstp = jnp.concatenate([dst, pad_d], axis=1).reshape(_NW, _NCHUNK, _CHUNK)
    z128 = jnp.zeros((_RPT, _DH), jnp.float32)
    ones = jnp.ones((_CHUNK, _DH), jnp.float32)

    degp = _sc_deg(dstp, z128, ones)
    yl1, yr1 = _proj2(x, Wl1, Wr1, b1, _DH)
    parts1 = _sc_segsum(_DH)(yl1, srcp, dstp, z128)
    yl2, yr2 = _combine_proj(parts1, degp, yr1, Wl2, Wr2, b2, _DH)
    parts2 = _sc_segsum(_DH)(yl2, srcp, dstp, z128)
    wpad = jnp.zeros((_DH - _DOUT, _DH), jnp.float32)
    Wl3p = jnp.concatenate([Wl3, wpad], axis=0)
    Wr3p = jnp.concatenate([Wr3, wpad], axis=0)
    b3p = jnp.concatenate([b3, jnp.zeros((_DH - _DOUT,), jnp.float32)])
    yl3, yr3p = _combine_proj(parts2, degp, yr2, Wl3p, Wr3p, b3p, _DH)
    yr3 = yr3p[:, :_DOUT]
    parts3 = _sc_segsum(_DH)(yl3, srcp, dstp, z128)
    out = _finalize(parts3, degp, yr3)
    return out
